# Initial kernel scaffold; baseline (speedup 1.0000x reference)
import jax
import jax.numpy as jnp
from jax.experimental import pallas as pl


def kernel(x, edge_index, W1, b1, W2, b2):
    raise NotImplementedError("write your pallas kernel here")


# speedup vs baseline: 10.1685x; 10.1685x over previous
import functools

import jax
import jax.numpy as jnp
from jax import lax
from jax.experimental import pallas as pl
from jax.experimental.pallas import tpu as pltpu
from jax.experimental.pallas import tpu_sc as plsc

N = 10000
D = 128
NC = 2
NS = 16
NW = NC * NS
CH = 128

ACC_ROWS = 10112
SEG = ACC_ROWS // NS
COUNT_PAD = 10240
CSEG = COUNT_PAD // NS


def _pad_edges(src, dst):
    e = src.shape[0]
    per = NW * CH
    j = -(-e // per)
    pad = j * per - e
    srcp = jnp.concatenate([src, jnp.zeros((pad,), jnp.int32)])
    dstp = jnp.concatenate([dst, jnp.full((pad,), N, jnp.int32)])
    return srcp.reshape(NW, j, CH), dstp.reshape(NW, j, CH)


def _mesh():
    return plsc.VectorSubcoreMesh(core_axis_name="c", subcore_axis_name="s")


def _sc_degree(dstp):
    nj = dstp.shape[1]

    @functools.partial(
        pl.kernel,
        mesh=_mesh(),
        out_type=jax.ShapeDtypeStruct((NC, COUNT_PAD), jnp.float32),
        scratch_types=[
            pltpu.VMEM((CH,), jnp.float32),
            pltpu.VMEM((CH,), jnp.int32),
            pltpu.VMEM((CSEG,), jnp.float32),
            pltpu.VMEM_SHARED((COUNT_PAD,), jnp.float32),
        ],
    )
    def deg_k(dst_hbm, out_hbm, ones_v, idx_v, zrow_v, counts):
        cid = lax.axis_index("c")
        sid = lax.axis_index("s")
        wid = sid * NC + cid
        z16 = jnp.zeros((16,), jnp.float32)
        o16 = jnp.ones((16,), jnp.float32)
        for k in range(CH // 16):
            ones_v[pl.ds(k * 16, 16)] = o16

        def zb(i, c):
            zrow_v[pl.ds(i * 16, 16)] = z16
            return c

        lax.fori_loop(0, CSEG // 16, zb, 0)
        pltpu.sync_copy(zrow_v, counts.at[pl.ds(sid * CSEG, CSEG)])
        plsc.subcore_barrier()

        def body(j, c):
            pltpu.sync_copy(dst_hbm.at[wid, j], idx_v)
            pltpu.sync_copy(ones_v, counts.at[idx_v], add=True)
            return c

        lax.fori_loop(0, nj, body, 0)
        plsc.subcore_barrier()
        pltpu.sync_copy(counts.at[pl.ds(sid * CSEG, CSEG)],
                        out_hbm.at[cid, pl.ds(sid * CSEG, CSEG)])

    return deg_k(dstp)


def _sc_scatter(table, srcp, dstp):
    nj = srcp.shape[1]

    @functools.partial(
        pl.kernel,
        mesh=_mesh(),
        out_type=jax.ShapeDtypeStruct((NC, ACC_ROWS, D), jnp.float32),
        scratch_types=[
            pltpu.VMEM((CH,), jnp.int32),
            pltpu.VMEM((CH,), jnp.int32),
            pltpu.VMEM((CH, D), jnp.float32),
            pltpu.VMEM((CH, D), jnp.float32),
            pltpu.SemaphoreType.DMA,
            pltpu.VMEM_SHARED((ACC_ROWS, D), jnp.float32),
        ],
    )
    def scat_k(tab_hbm, src_hbm, dst_hbm, out_hbm,
               sidx, didx, rows, zstage, sem, acc):
        cid = lax.axis_index("c")
        sid = lax.axis_index("s")
        wid = sid * NC + cid
        z16 = jnp.zeros((16,), jnp.float32)

        def zrow(r, c):
            for k in range(D // 16):
                zstage[r, pl.ds(k * 16, 16)] = z16
            return c

        lax.fori_loop(0, CH, zrow, 0)
        base = sid * SEG
        nfull, rem = SEG // CH, SEG % CH
        for c in range(nfull):
            pltpu.sync_copy(zstage, acc.at[pl.ds(base + c * CH, CH)])
        if rem:
            pltpu.sync_copy(zstage.at[pl.ds(0, rem)],
                            acc.at[pl.ds(base + nfull * CH, rem)])
        plsc.subcore_barrier()

        def body(j, c):
            pltpu.sync_copy(src_hbm.at[wid, j], sidx)
            pltpu.sync_copy(dst_hbm.at[wid, j], didx)
            pltpu.async_copy(tab_hbm.at[sidx], rows, sem).wait()
            pltpu.sync_copy(rows, acc.at[didx], add=True)
            return c

        lax.fori_loop(0, nj, body, 0)
        plsc.subcore_barrier()
        pltpu.sync_copy(acc.at[pl.ds(base, SEG)],
                        out_hbm.at[cid, pl.ds(base, SEG)])

    return scat_k(table, srcp, dstp)


_BR = 1000


def _tc_scale_matmul(x, c0, c1, w1):
    def body(x_ref, c0_ref, c1_ref, w_ref, g_ref, dinv_ref):
        dinv = lax.rsqrt(c0_ref[...] + c1_ref[...] + 1.0)
        g_ref[...] = jnp.dot(x_ref[...] * dinv, w_ref[...],
                             preferred_element_type=jnp.float32)
        dinv_ref[...] = dinv

    n = x.shape[0]
    return pl.pallas_call(
        body,
        grid=(n // _BR,),
        in_specs=[
            pl.BlockSpec((_BR, D), lambda i: (i, 0)),
            pl.BlockSpec((_BR, 1), lambda i: (i, 0)),
            pl.BlockSpec((_BR, 1), lambda i: (i, 0)),
            pl.BlockSpec((D, D), lambda i: (0, 0)),
        ],
        out_specs=[
            pl.BlockSpec((_BR, D), lambda i: (i, 0)),
            pl.BlockSpec((_BR, 1), lambda i: (i, 0)),
        ],
        out_shape=[
            jax.ShapeDtypeStruct((n, D), jnp.float32),
            jax.ShapeDtypeStruct((n, 1), jnp.float32),
        ],
    )(x, c0, c1, w1)


def _tc_mid(p0, p1, g1, dinv, b1, w2):
    def body(p0_ref, p1_ref, g_ref, dinv_ref, b_ref, w_ref, o_ref):
        t = dinv_ref[...] * (p0_ref[...] + p1_ref[...] + g_ref[...]) + b_ref[...]
        r = jnp.maximum(t, 0.0)
        o_ref[...] = dinv_ref[...] * jnp.dot(r, w_ref[...],
                                             preferred_element_type=jnp.float32)

    n = g1.shape[0]
    return pl.pallas_call(
        body,
        grid=(n // _BR,),
        in_specs=[
            pl.BlockSpec((_BR, D), lambda i: (i, 0)),
            pl.BlockSpec((_BR, D), lambda i: (i, 0)),
            pl.BlockSpec((_BR, D), lambda i: (i, 0)),
            pl.BlockSpec((_BR, 1), lambda i: (i, 0)),
            pl.BlockSpec((1, D), lambda i: (0, 0)),
            pl.BlockSpec((D, D), lambda i: (0, 0)),
        ],
        out_specs=pl.BlockSpec((_BR, D), lambda i: (i, 0)),
        out_shape=jax.ShapeDtypeStruct((n, D), jnp.float32),
    )(p0, p1, g1, dinv, b1, w2)


def _tc_final(q0, q1, g2, dinv, b2):
    def body(q0_ref, q1_ref, g_ref, dinv_ref, b_ref, o_ref):
        o_ref[...] = dinv_ref[...] * (
            q0_ref[...] + q1_ref[...] + g_ref[...]) + b_ref[...]

    n = g2.shape[0]
    return pl.pallas_call(
        body,
        grid=(n // _BR,),
        in_specs=[
            pl.BlockSpec((_BR, D), lambda i: (i, 0)),
            pl.BlockSpec((_BR, D), lambda i: (i, 0)),
            pl.BlockSpec((_BR, D), lambda i: (i, 0)),
            pl.BlockSpec((_BR, 1), lambda i: (i, 0)),
            pl.BlockSpec((1, D), lambda i: (0, 0)),
        ],
        out_specs=pl.BlockSpec((_BR, D), lambda i: (i, 0)),
        out_shape=jax.ShapeDtypeStruct((n, D), jnp.float32),
    )(q0, q1, g2, dinv, b2)


def kernel(x, edge_index, W1, b1, W2, b2):
    ei = edge_index.astype(jnp.int32)
    srcp, dstp = _pad_edges(ei[0], ei[1])

    counts = _sc_degree(dstp)
    c0 = counts[0, :N].reshape(N, 1)
    c1 = counts[1, :N].reshape(N, 1)

    g1, dinv = _tc_scale_matmul(x, c0, c1, W1)
    p = _sc_scatter(g1, srcp, dstp)
    g2 = _tc_mid(p[0, :N], p[1, :N], g1, dinv, b1.reshape(1, D), W2)
    q = _sc_scatter(g2, srcp, dstp)
    return _tc_final(q[0, :N], q[1, :N], g2, dinv, b2.reshape(1, D))
